# Initial kernel scaffold; baseline (speedup 1.0000x reference)
import jax
import jax.numpy as jnp
from jax.experimental import pallas as pl


def kernel(atomic_numbers, emb_table):
    raise NotImplementedError("write your pallas kernel here")


# speedup vs baseline: 1.5281x; 1.5281x over previous
import functools

import jax
import jax.numpy as jnp
from jax import lax
from jax.experimental import pallas as pl
from jax.experimental.pallas import tpu as pltpu
from jax.experimental.pallas import tpu_sc as plsc

NUM_NODES = 100000
VOCAB = 119
EMB_DIM = 128

NC = 2
NS = 16
NW = NC * NS

CB = 3200
SUB = 400
NSUB = CB // SUB


def _emb_body(idx_hbm, table_hbm, out_hbm, idx_v, rows_v, sem):
    wid = lax.axis_index("s") * NC + lax.axis_index("c")
    base = pl.multiple_of(jnp.minimum(wid * CB, NUM_NODES - CB), 8)
    pltpu.sync_copy(idx_hbm.at[pl.ds(base, CB)], idx_v)

    def step(j, carry):
        pltpu.async_copy(
            table_hbm.at[idx_v.at[pl.ds(j * SUB, SUB)]], rows_v, sem
        ).wait()
        pltpu.sync_copy(rows_v, out_hbm.at[pl.ds(base + j * SUB, SUB)])
        return carry

    lax.fori_loop(0, NSUB, step, 0)


@functools.partial(jax.jit, static_argnums=())
def _emb_lookup(atomic_numbers, emb_table):
    mesh = plsc.VectorSubcoreMesh(core_axis_name="c", subcore_axis_name="s")
    fn = functools.partial(
        pl.kernel,
        mesh=mesh,
        out_type=jax.ShapeDtypeStruct((NUM_NODES, EMB_DIM), jnp.float32),
        scratch_types=[
            pltpu.VMEM((CB,), jnp.int32),
            pltpu.VMEM((SUB, EMB_DIM), jnp.float32),
            pltpu.SemaphoreType.DMA,
        ],
    )(_emb_body)
    return fn(atomic_numbers, emb_table)


def kernel(atomic_numbers, emb_table):
    out = _emb_lookup(atomic_numbers.astype(jnp.int32), emb_table)
    return (out, out)
